# Initial kernel scaffold; baseline (speedup 1.0000x reference)
import jax
import jax.numpy as jnp
from jax.experimental import pallas as pl


def kernel(pred, target):
    raise NotImplementedError("write your pallas kernel here")


# speedup vs baseline: 140.6432x; 140.6432x over previous
import functools

import jax
import jax.numpy as jnp
from jax import lax
from jax.experimental import pallas as pl
from jax.experimental.pallas import tpu as pltpu, tpu_sc as plsc

NB = 1024
C = 19
NT = 32
PIX = 1048576
TPIX = PIX // NT
S = 4096
HSZ = 4 * C * NB

_mesh = plsc.VectorSubcoreMesh(core_axis_name="c", subcore_axis_name="s")


@functools.partial(
    pl.kernel,
    mesh=_mesh,
    compiler_params=pltpu.CompilerParams(needs_layout_passes=False),
    out_type=jax.ShapeDtypeStruct((NT, HSZ), jnp.float32),
    scratch_types=[
        pltpu.VMEM((HSZ,), jnp.float32),
        pltpu.VMEM((S,), jnp.int32),
        pltpu.VMEM((S,), jnp.float32),
    ],
)
def _sc_hist(pred_hbm, tgt_hbm, out_hbm, hist, lab_v, p_v):
    wid = lax.axis_index("s") * 2 + lax.axis_index("c")
    b = wid // 8
    part = wid % 8
    col0 = part * TPIX

    def zero_body(i, _):
        hist[pl.ds(i * 16, 16)] = jnp.zeros((16,), jnp.float32)
        return 0

    lax.fori_loop(0, HSZ // 16, zero_body, 0)

    ones = jnp.ones((16,), jnp.float32)

    def sub_body(si, _):
        pltpu.sync_copy(tgt_hbm.at[pl.ds(wid * TPIX + si * S, S)], lab_v)

        def c_body(c, _):
            pltpu.sync_copy(
                pred_hbm.at[b * C + c, pl.ds(col0 + si * S, S)], p_v)
            cNB = c * NB

            def v_body(i, _):
                lab = lab_v[pl.ds(i * 16, 16)]
                p = p_v[pl.ds(i * 16, 16)]
                hit = lab == c
                valid = lab != 0
                e = jnp.where(hit, 1.0 - p, p)
                t = e * jnp.float32(NB)
                bin_i = jnp.minimum(t.astype(jnp.int32), NB - 1)
                frac = t - bin_i.astype(jnp.float32)
                idx = bin_i + cNB
                hitmask = valid & hit
                plsc.addupdate_scatter(hist, [idx], ones, mask=valid)
                plsc.addupdate_scatter(hist, [idx + C * NB], frac, mask=valid)
                plsc.addupdate_scatter(hist, [idx + 2 * C * NB], ones, mask=hitmask)
                plsc.addupdate_scatter(hist, [idx + 3 * C * NB], frac, mask=hitmask)
                return 0

            lax.fori_loop(0, S // 16, v_body, 0)
            return 0

        lax.fori_loop(0, C, c_body, 0)
        return 0

    lax.fori_loop(0, TPIX // S, sub_body, 0)
    pltpu.sync_copy(hist, out_hbm.at[wid])


def _tc_post_body(h_ref, o_ref):
    h = jnp.sum(h_ref[...], axis=0)
    cnt_all = h[0:C]
    frac_all = h[C:2 * C]
    cnt_hit = h[2 * C:3 * C]
    frac_hit = h[3 * C:4 * C]
    src = lax.broadcasted_iota(jnp.int32, (NB, NB), 0)
    dst = lax.broadcasted_iota(jnp.int32, (NB, NB), 1)
    above = (src > dst).astype(jnp.float32)
    g_above = lax.dot(cnt_all, above, precision=lax.Precision.HIGHEST)
    h_above = lax.dot(cnt_hit, above, precision=lax.Precision.HIGHEST)
    gbar = g_above + frac_all
    hbar = h_above + frac_hit
    gts = jnp.sum(cnt_hit, axis=1, keepdims=True)
    inter = gts - hbar
    union = gts + gbar - hbar
    jac = 1.0 - inter / jnp.maximum(union, jnp.float32(1e-30))
    losses = jnp.sum(jac, axis=1) * jnp.float32(1.0 / NB)
    w = (gts[:, 0] > 0).astype(jnp.float32)
    loss = jnp.sum(losses * w) / jnp.sum(w)
    o_ref[0, 0] = loss


def kernel(pred, target):
    B, c, H, W = pred.shape
    pred_flat = pred.reshape(B * c, H * W)
    tgt_flat = target.reshape(B * H * W)
    hists = _sc_hist(pred_flat, tgt_flat)
    hists3 = hists.reshape(NT, 4 * C, NB)
    out = pl.pallas_call(
        _tc_post_body,
        out_shape=jax.ShapeDtypeStruct((1, 1), jnp.float32),
        out_specs=pl.BlockSpec(memory_space=pltpu.SMEM),
    )(hists3)
    return out[0, 0]


# speedup vs baseline: 177.2282x; 1.2601x over previous
import functools

import jax
import jax.numpy as jnp
from jax import lax
from jax.experimental import pallas as pl
from jax.experimental.pallas import tpu as pltpu, tpu_sc as plsc

NB = 1024
C = 19
NT = 32
PIX = 1048576
TPIX = PIX // NT
S = 2048
CNB = C * NB
HSZ = 4 * CNB
K = float(NB * (1.0 - 2.0 ** -22))

_mesh = plsc.VectorSubcoreMesh(core_axis_name="c", subcore_axis_name="s")


@functools.partial(
    pl.kernel,
    mesh=_mesh,
    compiler_params=pltpu.CompilerParams(needs_layout_passes=False),
    out_type=jax.ShapeDtypeStruct((NT, HSZ), jnp.float32),
    scratch_types=[
        pltpu.VMEM((HSZ,), jnp.float32),
        pltpu.VMEM((S,), jnp.int32),
        pltpu.VMEM((C, S), jnp.float32),
    ],
)
def _sc_hist(pred_hbm, tgt_hbm, out_hbm, hist, lab_v, p_v):
    wid = lax.axis_index("s") * 2 + lax.axis_index("c")
    b = wid // 8
    part = wid % 8
    col0 = part * TPIX

    def zero_body(i, _):
        hist[pl.ds(i * 16, 16)] = jnp.zeros((16,), jnp.float32)
        return 0

    lax.fori_loop(0, HSZ // 16, zero_body, 0)

    ones = jnp.ones((16,), jnp.float32)

    def sub_body(si, _):
        pltpu.sync_copy(tgt_hbm.at[pl.ds(wid * TPIX + si * S, S)], lab_v)
        pltpu.sync_copy(
            pred_hbm.at[b, :, pl.ds(col0 + si * S, S)], p_v)

        def v_body(i, _):
            lab = lab_v[pl.ds(i * 16, 16)]
            valid = lab != 0
            for c in range(C):
                p = p_v[c, pl.ds(i * 16, 16)]
                hit = lab == c
                t0 = p * K
                t = jnp.where(hit, K - t0, t0)
                bin_i = t.astype(jnp.int32)
                frac = t - bin_i.astype(jnp.float32)
                idx = bin_i + jnp.where(hit, CNB, 0)
                plsc.addupdate_scatter(
                    hist.at[pl.ds(c * NB, CNB + NB)], [idx], ones, mask=valid)
                plsc.addupdate_scatter(
                    hist.at[pl.ds(2 * CNB + c * NB, CNB + NB)], [idx], frac,
                    mask=valid)
            return 0

        lax.fori_loop(0, S // 16, v_body, 0)
        return 0

    lax.fori_loop(0, TPIX // S, sub_body, 0)
    pltpu.sync_copy(hist, out_hbm.at[wid])


def _tc_post_body(h_ref, o_ref):
    h = jnp.sum(h_ref[...], axis=0)
    cnt_hit = h[C:2 * C]
    cnt_all = h[0:C] + cnt_hit
    frac_hit = h[3 * C:4 * C]
    frac_all = h[2 * C:3 * C] + frac_hit
    src = lax.broadcasted_iota(jnp.int32, (NB, NB), 0)
    dst = lax.broadcasted_iota(jnp.int32, (NB, NB), 1)
    above = (src > dst).astype(jnp.float32)
    g_above = lax.dot(cnt_all, above, precision=lax.Precision.HIGHEST)
    h_above = lax.dot(cnt_hit, above, precision=lax.Precision.HIGHEST)
    gbar = g_above + frac_all
    hbar = h_above + frac_hit
    gts = jnp.sum(cnt_hit, axis=1, keepdims=True)
    inter = gts - hbar
    union = gts + gbar - hbar
    jac = 1.0 - inter / jnp.maximum(union, jnp.float32(1e-30))
    losses = jnp.sum(jac, axis=1) * jnp.float32(1.0 / NB)
    w = (gts[:, 0] > 0).astype(jnp.float32)
    loss = jnp.sum(losses * w) / jnp.sum(w)
    o_ref[0, 0] = loss


def kernel(pred, target):
    B, c, H, W = pred.shape
    pred_flat = pred.reshape(B, c, H * W)
    tgt_flat = target.reshape(B * H * W)
    hists = _sc_hist(pred_flat, tgt_flat)
    hists3 = hists.reshape(NT, 4 * C, NB)
    out = pl.pallas_call(
        _tc_post_body,
        out_shape=jax.ShapeDtypeStruct((1, 1), jnp.float32),
        out_specs=pl.BlockSpec(memory_space=pltpu.SMEM),
    )(hists3)
    return out[0, 0]


# speedup vs baseline: 196.0842x; 1.1064x over previous
import functools

import jax
import jax.numpy as jnp
from jax import lax
from jax.experimental import pallas as pl
from jax.experimental.pallas import tpu as pltpu, tpu_sc as plsc

NB = 1024
C = 19
NT = 32
PIX = 1048576
TPIX = PIX // NT
S = 2048
CNB = C * NB
HSZ = 4 * CNB
K = float(NB * (1.0 - 2.0 ** -22))

_mesh = plsc.VectorSubcoreMesh(core_axis_name="c", subcore_axis_name="s")


@functools.partial(
    pl.kernel,
    mesh=_mesh,
    compiler_params=pltpu.CompilerParams(needs_layout_passes=False),
    out_type=jax.ShapeDtypeStruct((NT, HSZ), jnp.float32),
    scratch_types=[
        pltpu.VMEM((HSZ,), jnp.float32),
        pltpu.VMEM((S,), jnp.int32),
        pltpu.VMEM((C, S), jnp.float32),
    ],
)
def _sc_hist(pred_hbm, tgt_hbm, out_hbm, hist, lab_v, p_v):
    wid = lax.axis_index("s") * 2 + lax.axis_index("c")
    b = wid // 8
    part = wid % 8
    col0 = part * TPIX

    def zero_body(i, _):
        hist[pl.ds(i * 16, 16)] = jnp.zeros((16,), jnp.float32)
        return 0

    lax.fori_loop(0, HSZ // 16, zero_body, 0)

    ones = jnp.ones((16,), jnp.float32)

    def sub_body(si, _):
        pltpu.sync_copy(tgt_hbm.at[pl.ds(wid * TPIX + si * S, S)], lab_v)
        pltpu.sync_copy(
            pred_hbm.at[b, :, pl.ds(col0 + si * S, S)], p_v)

        lane = lax.iota(jnp.int32, 16)
        neg_ones = -ones

        def v_body(i, _):
            lab = lab_v[pl.ds(i * 16, 16)]
            valid = lab != 0
            for c in range(C):
                p = p_v[c, pl.ds(i * 16, 16)]
                t = p * K
                bin_i = t.astype(jnp.int32)
                frac = t - bin_i.astype(jnp.float32)
                plsc.addupdate_scatter(
                    hist.at[pl.ds(c * NB, NB)], [bin_i], ones, mask=valid)
                plsc.addupdate_scatter(
                    hist.at[pl.ds(CNB + c * NB, NB)], [bin_i], frac,
                    mask=valid)
            pix = lane + i * 16
            p_lab = plsc.load_gather(p_v, [lab, pix])
            t_old = p_lab * K
            bo = t_old.astype(jnp.int32)
            frac_old = t_old - bo.astype(jnp.float32)
            lab_nb = lab * NB
            idx_old = lab_nb + bo
            t_new = K - t_old
            bn = t_new.astype(jnp.int32)
            frac_new = t_new - bn.astype(jnp.float32)
            idx_new = lab_nb + bn
            plsc.addupdate_scatter(
                hist.at[pl.ds(0, CNB)], [idx_old], neg_ones, mask=valid)
            plsc.addupdate_scatter(
                hist.at[pl.ds(CNB, CNB)], [idx_old], -frac_old, mask=valid)
            plsc.addupdate_scatter(
                hist.at[pl.ds(0, CNB)], [idx_new], ones, mask=valid)
            plsc.addupdate_scatter(
                hist.at[pl.ds(CNB, CNB)], [idx_new], frac_new, mask=valid)
            plsc.addupdate_scatter(
                hist.at[pl.ds(2 * CNB, CNB)], [idx_new], ones, mask=valid)
            plsc.addupdate_scatter(
                hist.at[pl.ds(3 * CNB, CNB)], [idx_new], frac_new, mask=valid)
            return 0

        lax.fori_loop(0, S // 16, v_body, 0)
        return 0

    lax.fori_loop(0, TPIX // S, sub_body, 0)
    pltpu.sync_copy(hist, out_hbm.at[wid])


def _tc_post_body(h_ref, o_ref):
    h = jnp.sum(h_ref[...], axis=0)
    cnt_all = h[0:C]
    frac_all = h[C:2 * C]
    cnt_hit = h[2 * C:3 * C]
    frac_hit = h[3 * C:4 * C]
    src = lax.broadcasted_iota(jnp.int32, (NB, NB), 0)
    dst = lax.broadcasted_iota(jnp.int32, (NB, NB), 1)
    above = (src > dst).astype(jnp.float32)
    g_above = lax.dot(cnt_all, above, precision=lax.Precision.HIGHEST)
    h_above = lax.dot(cnt_hit, above, precision=lax.Precision.HIGHEST)
    gbar = g_above + frac_all
    hbar = h_above + frac_hit
    gts = jnp.sum(cnt_hit, axis=1, keepdims=True)
    inter = gts - hbar
    union = gts + gbar - hbar
    jac = 1.0 - inter / jnp.maximum(union, jnp.float32(1e-30))
    losses = jnp.sum(jac, axis=1) * jnp.float32(1.0 / NB)
    w = (gts[:, 0] > 0).astype(jnp.float32)
    loss = jnp.sum(losses * w) / jnp.sum(w)
    o_ref[0, 0] = loss


def kernel(pred, target):
    B, c, H, W = pred.shape
    pred_flat = pred.reshape(B, c, H * W)
    tgt_flat = target.reshape(B * H * W)
    hists = _sc_hist(pred_flat, tgt_flat)
    hists3 = hists.reshape(NT, 4 * C, NB)
    out = pl.pallas_call(
        _tc_post_body,
        out_shape=jax.ShapeDtypeStruct((1, 1), jnp.float32),
        out_specs=pl.BlockSpec(memory_space=pltpu.SMEM),
    )(hists3)
    return out[0, 0]


# speedup vs baseline: 373.7656x; 1.9061x over previous
import functools

import jax
import jax.numpy as jnp
from jax import lax
from jax.experimental import pallas as pl
from jax.experimental.pallas import tpu as pltpu, tpu_sc as plsc

NB = 1024
C = 19
NT = 32
PIX = 1048576
TPIX = PIX // NT
S = 2048
CNB = C * NB
HSZ = 4 * CNB
K = float(NB * (1.0 - 2.0 ** -22))

_mesh = plsc.VectorSubcoreMesh(core_axis_name="c", subcore_axis_name="s")


@functools.partial(
    pl.kernel,
    mesh=_mesh,
    compiler_params=pltpu.CompilerParams(needs_layout_passes=False),
    out_type=jax.ShapeDtypeStruct((NT, HSZ), jnp.float32),
    scratch_types=[
        pltpu.VMEM((HSZ,), jnp.float32),
        pltpu.VMEM((S,), jnp.int32),
        pltpu.VMEM((C, S), jnp.float32),
    ],
)
def _sc_hist(pred_hbm, tgt_hbm, out_hbm, hist, lab_v, p_v):
    wid = lax.axis_index("s") * 2 + lax.axis_index("c")
    b = wid // 8
    part = wid % 8
    col0 = part * TPIX

    def zero_body(i, _):
        hist[pl.ds(i * 16, 16)] = jnp.zeros((16,), jnp.float32)
        return 0

    lax.fori_loop(0, HSZ // 16, zero_body, 0)

    ones = jnp.ones((16,), jnp.float32)

    def sub_body(si, _):
        pltpu.sync_copy(tgt_hbm.at[pl.ds(wid * TPIX + si * S, S)], lab_v)
        pltpu.sync_copy(
            pred_hbm.at[b, :, pl.ds(col0 + si * S, S)], p_v)

        lane = lax.iota(jnp.int32, 16)
        neg_ones = -ones

        @plsc.parallel_loop(0, S // 16, unroll=2)
        def v_body(i):
            lab = lab_v[pl.ds(i * 16, 16)]
            valid = lab != 0
            for c in range(C):
                p = p_v[c, pl.ds(i * 16, 16)]
                t = p * K
                bin_i = t.astype(jnp.int32)
                frac = t - bin_i.astype(jnp.float32)
                plsc.addupdate_scatter(
                    hist.at[pl.ds(c * NB, NB)], [bin_i], ones, mask=valid)
                plsc.addupdate_scatter(
                    hist.at[pl.ds(CNB + c * NB, NB)], [bin_i], frac,
                    mask=valid)
            pix = lane + i * 16
            p_lab = plsc.load_gather(p_v, [lab, pix])
            t_old = p_lab * K
            bo = t_old.astype(jnp.int32)
            frac_old = t_old - bo.astype(jnp.float32)
            lab_nb = lab * NB
            idx_old = lab_nb + bo
            t_new = K - t_old
            bn = t_new.astype(jnp.int32)
            frac_new = t_new - bn.astype(jnp.float32)
            idx_new = lab_nb + bn
            plsc.addupdate_scatter(
                hist.at[pl.ds(0, CNB)], [idx_old], neg_ones, mask=valid)
            plsc.addupdate_scatter(
                hist.at[pl.ds(CNB, CNB)], [idx_old], -frac_old, mask=valid)
            plsc.addupdate_scatter(
                hist.at[pl.ds(0, CNB)], [idx_new], ones, mask=valid)
            plsc.addupdate_scatter(
                hist.at[pl.ds(CNB, CNB)], [idx_new], frac_new, mask=valid)
            plsc.addupdate_scatter(
                hist.at[pl.ds(2 * CNB, CNB)], [idx_new], ones, mask=valid)
            plsc.addupdate_scatter(
                hist.at[pl.ds(3 * CNB, CNB)], [idx_new], frac_new, mask=valid)

        return 0

    lax.fori_loop(0, TPIX // S, sub_body, 0)
    pltpu.sync_copy(hist, out_hbm.at[wid])


def _tc_post_body(h_ref, o_ref):
    h = jnp.sum(h_ref[...], axis=0)
    cnt_all = h[0:C]
    frac_all = h[C:2 * C]
    cnt_hit = h[2 * C:3 * C]
    frac_hit = h[3 * C:4 * C]
    src = lax.broadcasted_iota(jnp.int32, (NB, NB), 0)
    dst = lax.broadcasted_iota(jnp.int32, (NB, NB), 1)
    above = (src > dst).astype(jnp.float32)
    g_above = lax.dot(cnt_all, above, precision=lax.Precision.HIGHEST)
    h_above = lax.dot(cnt_hit, above, precision=lax.Precision.HIGHEST)
    gbar = g_above + frac_all
    hbar = h_above + frac_hit
    gts = jnp.sum(cnt_hit, axis=1, keepdims=True)
    inter = gts - hbar
    union = gts + gbar - hbar
    jac = 1.0 - inter / jnp.maximum(union, jnp.float32(1e-30))
    losses = jnp.sum(jac, axis=1) * jnp.float32(1.0 / NB)
    w = (gts[:, 0] > 0).astype(jnp.float32)
    loss = jnp.sum(losses * w) / jnp.sum(w)
    o_ref[0, 0] = loss


def kernel(pred, target):
    B, c, H, W = pred.shape
    pred_flat = pred.reshape(B, c, H * W)
    tgt_flat = target.reshape(B * H * W)
    hists = _sc_hist(pred_flat, tgt_flat)
    hists3 = hists.reshape(NT, 4 * C, NB)
    out = pl.pallas_call(
        _tc_post_body,
        out_shape=jax.ShapeDtypeStruct((1, 1), jnp.float32),
        out_specs=pl.BlockSpec(memory_space=pltpu.SMEM),
    )(hists3)
    return out[0, 0]


# speedup vs baseline: 597.3762x; 1.5983x over previous
import functools

import jax
import jax.numpy as jnp
from jax import lax
from jax.experimental import pallas as pl
from jax.experimental.pallas import tpu as pltpu, tpu_sc as plsc

NB = 512
C = 19
NT = 32
PIX = 1048576
TPIX = PIX // NT
ROWS = 8
S = ROWS * 512
NSUB = TPIX // S
CNB = C * NB
HSZ = 4 * CNB
K = float(NB * (1.0 - 2.0 ** -22))

_mesh = plsc.VectorSubcoreMesh(core_axis_name="c", subcore_axis_name="s")


@functools.partial(
    pl.kernel,
    mesh=_mesh,
    compiler_params=pltpu.CompilerParams(needs_layout_passes=False),
    out_type=jax.ShapeDtypeStruct((NT, HSZ), jnp.float32),
    scratch_types=[
        pltpu.VMEM((HSZ,), jnp.float32),
        pltpu.VMEM((ROWS, 512), jnp.int32),
        pltpu.VMEM((C, ROWS, 512), jnp.float32),
    ],
)
def _sc_hist(pred_hbm, tgt_hbm, out_hbm, hist, lab_v, p_v):
    wid = lax.axis_index("s") * 2 + lax.axis_index("c")
    b = wid // 8
    part = wid % 8
    row0 = part * 64

    def zero_body(i, _):
        hist[pl.ds(i * 16, 16)] = jnp.zeros((16,), jnp.float32)
        return 0

    lax.fori_loop(0, HSZ // 16, zero_body, 0)

    ones = jnp.ones((16,), jnp.float32)
    lane = lax.iota(jnp.int32, 16)
    neg_ones = -ones

    def sub_body(si, _):
        r0 = row0 + si * ROWS
        pltpu.sync_copy(tgt_hbm.at[b, pl.ds(r0, ROWS), :], lab_v)
        pltpu.sync_copy(pred_hbm.at[pl.ds(b * C, C), pl.ds(r0, ROWS), :], p_v)

        @plsc.parallel_loop(0, S // 16, unroll=2)
        def v_body(i):
            j = i // 32
            k = (i % 32) * 16
            lab = lab_v[j, pl.ds(k, 16)]
            valid = lab != 0
            for c in range(C):
                p = p_v[c, j, pl.ds(k, 16)]
                t = p * K
                bin_i = t.astype(jnp.int32)
                frac = t - bin_i.astype(jnp.float32)
                plsc.addupdate_scatter(
                    hist.at[pl.ds(c * NB, NB)], [bin_i], ones, mask=valid)
                plsc.addupdate_scatter(
                    hist.at[pl.ds(CNB + c * NB, NB)], [bin_i], frac,
                    mask=valid)
            jv = jnp.full((16,), j, jnp.int32)
            kv = lane + k
            p_lab = plsc.load_gather(p_v, [lab, jv, kv])
            t_old = p_lab * K
            bo = t_old.astype(jnp.int32)
            frac_old = t_old - bo.astype(jnp.float32)
            lab_nb = lab * NB
            idx_old = lab_nb + bo
            t_new = K - t_old
            bn = t_new.astype(jnp.int32)
            frac_new = t_new - bn.astype(jnp.float32)
            idx_new = lab_nb + bn
            plsc.addupdate_scatter(
                hist.at[pl.ds(0, CNB)], [idx_old], neg_ones, mask=valid)
            plsc.addupdate_scatter(
                hist.at[pl.ds(CNB, CNB)], [idx_old], -frac_old, mask=valid)
            plsc.addupdate_scatter(
                hist.at[pl.ds(0, CNB)], [idx_new], ones, mask=valid)
            plsc.addupdate_scatter(
                hist.at[pl.ds(CNB, CNB)], [idx_new], frac_new, mask=valid)
            plsc.addupdate_scatter(
                hist.at[pl.ds(2 * CNB, CNB)], [idx_new], ones, mask=valid)
            plsc.addupdate_scatter(
                hist.at[pl.ds(3 * CNB, CNB)], [idx_new], frac_new, mask=valid)

        return 0

    lax.fori_loop(0, NSUB, sub_body, 0)
    pltpu.sync_copy(hist, out_hbm.at[wid])


def _tc_post_body(h_ref, o_ref):
    h = jnp.sum(h_ref[...], axis=0)
    cnt_all = h[0:C]
    frac_all = h[C:2 * C]
    cnt_hit = h[2 * C:3 * C]
    frac_hit = h[3 * C:4 * C]
    src = lax.broadcasted_iota(jnp.int32, (NB, NB), 0)
    dst = lax.broadcasted_iota(jnp.int32, (NB, NB), 1)
    above = (src > dst).astype(jnp.float32)
    g_above = lax.dot(cnt_all, above, precision=lax.Precision.HIGHEST)
    h_above = lax.dot(cnt_hit, above, precision=lax.Precision.HIGHEST)
    gbar = g_above + frac_all
    hbar = h_above + frac_hit
    gts = jnp.sum(cnt_hit, axis=1, keepdims=True)
    inter = gts - hbar
    union = gts + gbar - hbar
    jac = 1.0 - inter / jnp.maximum(union, jnp.float32(1e-30))
    losses = jnp.sum(jac, axis=1) * jnp.float32(1.0 / NB)
    w = (gts[:, 0] > 0).astype(jnp.float32)
    loss = jnp.sum(losses * w) / jnp.sum(w)
    o_ref[0, 0] = loss


def kernel(pred, target):
    B, c, H, W = pred.shape
    pred_flat = pred.reshape(B * c, H, W)
    hists = _sc_hist(pred_flat, target)
    hists3 = hists.reshape(NT, 4 * C, NB)
    out = pl.pallas_call(
        _tc_post_body,
        out_shape=jax.ShapeDtypeStruct((1, 1), jnp.float32),
        out_specs=pl.BlockSpec(memory_space=pltpu.SMEM),
    )(hists3)
    return out[0, 0]


# speedup vs baseline: 614.3620x; 1.0284x over previous
import functools

import jax
import jax.numpy as jnp
from jax import lax
from jax.experimental import pallas as pl
from jax.experimental.pallas import tpu as pltpu, tpu_sc as plsc

NB = 512
C = 19
NT = 32
PIX = 1048576
TPIX = PIX // NT
ROWS = 8
S = ROWS * 512
NSUB = TPIX // S
CNB = C * NB
HSZ = 4 * CNB
K = float(NB * (1.0 - 2.0 ** -22))

_mesh = plsc.VectorSubcoreMesh(core_axis_name="c", subcore_axis_name="s")


@functools.partial(
    pl.kernel,
    mesh=_mesh,
    compiler_params=pltpu.CompilerParams(needs_layout_passes=False),
    out_type=jax.ShapeDtypeStruct((NT, HSZ), jnp.float32),
    scratch_types=[
        pltpu.VMEM((HSZ,), jnp.float32),
        pltpu.VMEM((ROWS, 512), jnp.int32),
        pltpu.VMEM((C, ROWS, 512), jnp.float32),
    ],
)
def _sc_hist(pred_hbm, tgt_hbm, out_hbm, hist, lab_v, p_v):
    wid = lax.axis_index("s") * 2 + lax.axis_index("c")
    b = wid // 8
    part = wid % 8
    row0 = part * 64

    def zero_body(i, _):
        hist[pl.ds(i * 16, 16)] = jnp.zeros((16,), jnp.float32)
        return 0

    lax.fori_loop(0, HSZ // 16, zero_body, 0)

    ones = jnp.ones((16,), jnp.float32)
    lane = lax.iota(jnp.int32, 16)
    neg_ones = -ones

    def sub_body(si, _):
        r0 = row0 + si * ROWS
        pltpu.sync_copy(tgt_hbm.at[b, pl.ds(r0, ROWS), :], lab_v)
        pltpu.sync_copy(pred_hbm.at[pl.ds(b * C, C), pl.ds(r0, ROWS), :], p_v)

        @plsc.parallel_loop(0, S // 16, unroll=2)
        def v_body(i):
            j = i // 32
            k = (i % 32) * 16
            lab = lab_v[j, pl.ds(k, 16)]
            valid = lab != 0
            for c in range(C):
                p = p_v[c, j, pl.ds(k, 16)]
                t = p * K
                bin_i = t.astype(jnp.int32)
                frac = t - bin_i.astype(jnp.float32)
                plsc.addupdate_scatter(
                    hist.at[pl.ds(c * NB, NB)], [bin_i], ones, mask=valid)
                plsc.addupdate_scatter(
                    hist.at[pl.ds(CNB + c * NB, NB)], [bin_i], frac,
                    mask=valid)
            jv = jnp.full((16,), j, jnp.int32)
            kv = lane + k
            p_lab = plsc.load_gather(p_v, [lab, jv, kv])
            t_old = p_lab * K
            bo = t_old.astype(jnp.int32)
            frac_old = t_old - bo.astype(jnp.float32)
            lab_nb = lab * NB
            idx_old = lab_nb + bo
            t_new = K - t_old
            bn = t_new.astype(jnp.int32)
            frac_new = t_new - bn.astype(jnp.float32)
            idx_new = lab_nb + bn
            plsc.addupdate_scatter(
                hist.at[pl.ds(0, CNB)], [idx_old], neg_ones, mask=valid)
            plsc.addupdate_scatter(
                hist.at[pl.ds(CNB, CNB)], [idx_old], -frac_old, mask=valid)
            plsc.addupdate_scatter(
                hist.at[pl.ds(0, CNB)], [idx_new], ones, mask=valid)
            plsc.addupdate_scatter(
                hist.at[pl.ds(CNB, CNB)], [idx_new], frac_new, mask=valid)
            plsc.addupdate_scatter(
                hist.at[pl.ds(2 * CNB, CNB)], [idx_new], ones, mask=valid)
            plsc.addupdate_scatter(
                hist.at[pl.ds(3 * CNB, CNB)], [idx_new], frac_new, mask=valid)

        return 0

    lax.fori_loop(0, NSUB, sub_body, 0)
    pltpu.sync_copy(hist, out_hbm.at[wid])


def _tc_post_body(h_ref, o_ref):
    h = jnp.sum(h_ref[...], axis=0).reshape(4 * C, NB)
    cnt_all = h[0:C]
    frac_all = h[C:2 * C]
    cnt_hit = h[2 * C:3 * C]
    frac_hit = h[3 * C:4 * C]
    src = lax.broadcasted_iota(jnp.int32, (NB, NB), 0)
    dst = lax.broadcasted_iota(jnp.int32, (NB, NB), 1)
    above = (src > dst).astype(jnp.float32)
    g_above = lax.dot(cnt_all, above, precision=lax.Precision.HIGHEST)
    h_above = lax.dot(cnt_hit, above, precision=lax.Precision.HIGHEST)
    gbar = g_above + frac_all
    hbar = h_above + frac_hit
    gts = jnp.sum(cnt_hit, axis=1, keepdims=True)
    inter = gts - hbar
    union = gts + gbar - hbar
    jac = 1.0 - inter / jnp.maximum(union, jnp.float32(1e-30))
    losses = jnp.sum(jac, axis=1) * jnp.float32(1.0 / NB)
    w = (gts[:, 0] > 0).astype(jnp.float32)
    loss = jnp.sum(losses * w) / jnp.sum(w)
    o_ref[0, 0] = loss


def kernel(pred, target):
    B, c, H, W = pred.shape
    pred_flat = pred.reshape(B * c, H, W)
    hists = _sc_hist(pred_flat, target)
    out = pl.pallas_call(
        _tc_post_body,
        out_shape=jax.ShapeDtypeStruct((1, 1), jnp.float32),
        out_specs=pl.BlockSpec(memory_space=pltpu.SMEM),
    )(hists)
    return out[0, 0]


# speedup vs baseline: 696.0063x; 1.1329x over previous
import functools

import jax
import jax.numpy as jnp
from jax import lax
from jax.experimental import pallas as pl
from jax.experimental.pallas import tpu as pltpu, tpu_sc as plsc

NB = 512
C = 19
NT = 32
PIX = 1048576
TPIX = PIX // NT
ROWS = 8
COLS = 128
S = ROWS * COLS
NSUB = TPIX // S
CNB = C * NB
HSZ = 4 * CNB
K = float(NB * (1.0 - 2.0 ** -22))

_mesh = plsc.VectorSubcoreMesh(core_axis_name="c", subcore_axis_name="s")


@functools.partial(
    pl.kernel,
    mesh=_mesh,
    compiler_params=pltpu.CompilerParams(needs_layout_passes=False),
    out_type=jax.ShapeDtypeStruct((NT, HSZ), jnp.float32),
    scratch_types=[
        pltpu.VMEM((HSZ,), jnp.float32),
        pltpu.VMEM((ROWS, COLS), jnp.int32),
        pltpu.VMEM((C, ROWS, COLS), jnp.float32),
        pltpu.VMEM((ROWS, COLS), jnp.int32),
        pltpu.VMEM((C, ROWS, COLS), jnp.float32),
        pltpu.SemaphoreType.DMA,
        pltpu.SemaphoreType.DMA,
    ],
)
def _sc_hist(pred_hbm, tgt_hbm, out_hbm, hist, lab_v0, p_v0, lab_v1, p_v1,
             sem0, sem1):
    wid = lax.axis_index("s") * 2 + lax.axis_index("c")
    b = wid // 8
    part = wid % 8
    row0 = part * 64

    def zero_body(i, _):
        hist[pl.ds(i * 16, 16)] = jnp.zeros((16,), jnp.float32)
        return 0

    lax.fori_loop(0, HSZ // 16, zero_body, 0)

    ones = jnp.ones((16,), jnp.float32)
    lane = lax.iota(jnp.int32, 16)
    neg_ones = -ones

    def start_copy(si, lab_v, p_v, sem):
        r0 = row0 + (si // 4) * ROWS
        c0 = (si % 4) * COLS
        pltpu.async_copy(
            tgt_hbm.at[b, pl.ds(r0, ROWS), pl.ds(c0, COLS)], lab_v, sem)
        pltpu.async_copy(
            pred_hbm.at[pl.ds(b * C, C), pl.ds(r0, ROWS), pl.ds(c0, COLS)],
            p_v, sem)

    def drain(sem, lab_v, p_v):
        pltpu.make_async_copy(
            tgt_hbm.at[0, pl.ds(0, ROWS), pl.ds(0, COLS)], lab_v, sem).wait()
        pltpu.make_async_copy(
            pred_hbm.at[pl.ds(0, C), pl.ds(0, ROWS), pl.ds(0, COLS)], p_v,
            sem).wait()

    def compute(lab_v, p_v):
        @plsc.parallel_loop(0, S // 16, unroll=2)
        def v_body(i):
            j = i // (COLS // 16)
            k = (i % (COLS // 16)) * 16
            lab = lab_v[j, pl.ds(k, 16)]
            valid = lab != 0
            for c in range(C):
                p = p_v[c, j, pl.ds(k, 16)]
                t = p * K
                bin_i = t.astype(jnp.int32)
                frac = t - bin_i.astype(jnp.float32)
                plsc.addupdate_scatter(
                    hist.at[pl.ds(c * NB, NB)], [bin_i], ones, mask=valid)
                plsc.addupdate_scatter(
                    hist.at[pl.ds(CNB + c * NB, NB)], [bin_i], frac,
                    mask=valid)
            jv = jnp.full((16,), j, jnp.int32)
            kv = lane + k
            p_lab = plsc.load_gather(p_v, [lab, jv, kv])
            t_old = p_lab * K
            bo = t_old.astype(jnp.int32)
            frac_old = t_old - bo.astype(jnp.float32)
            lab_nb = lab * NB
            idx_old = lab_nb + bo
            t_new = K - t_old
            bn = t_new.astype(jnp.int32)
            frac_new = t_new - bn.astype(jnp.float32)
            idx_new = lab_nb + bn
            plsc.addupdate_scatter(
                hist.at[pl.ds(0, CNB)], [idx_old], neg_ones, mask=valid)
            plsc.addupdate_scatter(
                hist.at[pl.ds(CNB, CNB)], [idx_old], -frac_old, mask=valid)
            plsc.addupdate_scatter(
                hist.at[pl.ds(0, CNB)], [idx_new], ones, mask=valid)
            plsc.addupdate_scatter(
                hist.at[pl.ds(CNB, CNB)], [idx_new], frac_new, mask=valid)
            plsc.addupdate_scatter(
                hist.at[pl.ds(2 * CNB, CNB)], [idx_new], ones, mask=valid)
            plsc.addupdate_scatter(
                hist.at[pl.ds(3 * CNB, CNB)], [idx_new], frac_new, mask=valid)

    start_copy(0, lab_v0, p_v0, sem0)

    def pair_body(q, _):
        start_copy(2 * q + 1, lab_v1, p_v1, sem1)
        drain(sem0, lab_v0, p_v0)
        compute(lab_v0, p_v0)

        @pl.when(q < NSUB // 2 - 1)
        def _():
            start_copy(2 * q + 2, lab_v0, p_v0, sem0)

        drain(sem1, lab_v1, p_v1)
        compute(lab_v1, p_v1)
        return 0

    lax.fori_loop(0, NSUB // 2, pair_body, 0)
    pltpu.sync_copy(hist, out_hbm.at[wid])


def _tc_post_body(h_ref, o_ref):
    h = jnp.sum(h_ref[...], axis=0).reshape(4 * C, NB)
    cnt_all = h[0:C]
    frac_all = h[C:2 * C]
    cnt_hit = h[2 * C:3 * C]
    frac_hit = h[3 * C:4 * C]
    src = lax.broadcasted_iota(jnp.int32, (NB, NB), 0)
    dst = lax.broadcasted_iota(jnp.int32, (NB, NB), 1)
    above = (src > dst).astype(jnp.float32)
    g_above = lax.dot(cnt_all, above, precision=lax.Precision.HIGHEST)
    h_above = lax.dot(cnt_hit, above, precision=lax.Precision.HIGHEST)
    gbar = g_above + frac_all
    hbar = h_above + frac_hit
    gts = jnp.sum(cnt_hit, axis=1, keepdims=True)
    inter = gts - hbar
    union = gts + gbar - hbar
    jac = 1.0 - inter / jnp.maximum(union, jnp.float32(1e-30))
    losses = jnp.sum(jac, axis=1) * jnp.float32(1.0 / NB)
    w = (gts[:, 0] > 0).astype(jnp.float32)
    loss = jnp.sum(losses * w) / jnp.sum(w)
    o_ref[0, 0] = loss


def kernel(pred, target):
    B, c, H, W = pred.shape
    pred_flat = pred.reshape(B * c, H, W)
    hists = _sc_hist(pred_flat, target)
    out = pl.pallas_call(
        _tc_post_body,
        out_shape=jax.ShapeDtypeStruct((1, 1), jnp.float32),
        out_specs=pl.BlockSpec(memory_space=pltpu.SMEM),
    )(hists)
    return out[0, 0]
